# baseline (speedup 1.0000x reference)
import functools

import jax
import jax.numpy as jnp
from jax import lax
from jax.experimental import pallas as pl
from jax.experimental.pallas import tpu as pltpu
from jax.experimental.pallas import tpu_sc as plsc

_B = 16384
_D = 64
_C = 100000
_CPAD = 100096
_NC = 2
_NS = 16
_NW = _NC * _NS
_RW = _B // _NW
_GROUPS = _RW // 16
_HROWS = (_B // 128) // _NS
_ZCHUNK = _CPAD // _NS


def _vsqrt16(x):
    i = plsc.bitcast(x, jnp.int32)
    y = plsc.bitcast(jnp.int32(0x5F3759DF) - (i >> 1), jnp.float32)
    h = 0.5 * x
    for _ in range(4):
        y = y * (1.5 - (h * y) * y)
    return x * y


def _body(feat_hbm, label_hbm, centers_hbm, out_hbm,
          lbl_d, lbl_h, ones_v, zeros_v, cent_v, feat_v, num_v,
          acc_v, sums_v, out_v, count_s, partials_s, sem):
    cid = lax.axis_index("c")
    sid = lax.axis_index("s")
    wid = sid * _NC + cid

    pltpu.sync_copy(label_hbm.at[pl.ds(wid * 4, 4)], lbl_d)
    copies = []
    for c in range(4):
        copies.append(pltpu.async_copy(
            centers_hbm.at[lbl_d.at[c]],
            cent_v.at[pl.ds(c * 128, 128)], sem))
    copies.append(pltpu.async_copy(
        feat_hbm.at[pl.ds(wid * _RW, _RW)], feat_v, sem))

    def _zbody(i, carry):
        zeros_v[pl.ds(i * 16, 16)] = jnp.zeros((16,), jnp.float32)
        return carry
    lax.fori_loop(0, _ZCHUNK // 16, _zbody, 0)
    for j in range(8):
        ones_v[pl.ds(j * 16, 16)] = jnp.ones((16,), jnp.float32)
    pltpu.sync_copy(zeros_v, count_s.at[pl.ds(sid * _ZCHUNK, _ZCHUNK)])
    plsc.subcore_barrier()

    pltpu.sync_copy(label_hbm.at[pl.ds(sid * _HROWS, _HROWS)], lbl_h)
    for j in range(_HROWS):
        pltpu.sync_copy(ones_v, count_s.at[lbl_h.at[j]], add=True)
    plsc.subcore_barrier()

    for c in range(4):
        pltpu.sync_copy(count_s.at[lbl_d.at[c]],
                        num_v.at[pl.ds(c * 128, 128)])
    for cp in copies:
        cp.wait()

    iota = lax.iota(jnp.int32, 16)

    def _gbody(g, acc):
        d2 = jnp.zeros((16,), jnp.float32)
        for k in range(16):
            r = g * 16 + k
            s = jnp.zeros((16,), jnp.float32)
            for c in range(_D // 16):
                fv = feat_v[r, pl.ds(c * 16, 16)]
                cv = cent_v[r, pl.ds(c * 16, 16)]
                t = fv - cv
                s = s + t * t
            d2 = jnp.where(iota == k, jnp.sum(s), d2)
        num16 = num_v[pl.ds(g * 16, 16)]
        return acc + _vsqrt16(d2) / num16

    acc = lax.fori_loop(0, _GROUPS, _gbody, jnp.zeros((16,), jnp.float32))

    acc_v[...] = acc
    pltpu.sync_copy(acc_v, partials_s.at[sid])
    plsc.subcore_barrier()

    @pl.when(sid == 0)
    def _():
        pltpu.sync_copy(partials_s, sums_v)
        tot = jnp.zeros((16,), jnp.float32)
        for i in range(_NS):
            tot = tot + sums_v[i]
        total = jnp.sum(tot)
        out_v[...] = jnp.full((16,), total, jnp.float32)
        pltpu.sync_copy(out_v, out_hbm.at[cid])


@jax.jit
def _center_loss_sc(feat, label_r, centers):
    mesh = plsc.VectorSubcoreMesh(core_axis_name="c", subcore_axis_name="s")
    f = pl.kernel(
        _body,
        out_type=jax.ShapeDtypeStruct((_NC, 16), jnp.float32),
        mesh=mesh,
        scratch_types=[
            pltpu.VMEM((4, 128), jnp.int32),
            pltpu.VMEM((_HROWS, 128), jnp.int32),
            pltpu.VMEM((128,), jnp.float32),
            pltpu.VMEM((_ZCHUNK,), jnp.float32),
            pltpu.VMEM((_RW, _D), jnp.float32),
            pltpu.VMEM((_RW, _D), jnp.float32),
            pltpu.VMEM((_RW,), jnp.float32),
            pltpu.VMEM((16,), jnp.float32),
            pltpu.VMEM((_NS, 16), jnp.float32),
            pltpu.VMEM((16,), jnp.float32),
            pltpu.VMEM_SHARED((_CPAD,), jnp.float32),
            pltpu.VMEM_SHARED((_NS, 16), jnp.float32),
            pltpu.SemaphoreType.DMA,
        ],
        compiler_params=pltpu.CompilerParams(
            needs_layout_passes=False, use_tc_tiling_on_sc=False),
    )
    return f(feat, label_r, centers)


def kernel(feat, label, centers):
    label_r = label.reshape(_B // 128, 128)
    out = _center_loss_sc(feat, label_r, centers)
    return (out[0, 0] + out[1, 0]) / jnp.float32(_B)


# speedup vs baseline: 1.4084x; 1.4084x over previous
import jax
import jax.numpy as jnp
from jax import lax
from jax.experimental import pallas as pl
from jax.experimental.pallas import tpu as pltpu
from jax.experimental.pallas import tpu_sc as plsc

_B = 16384
_D = 64
_C = 100000
_CPAD = 100096
_NC = 2
_NS = 16
_NW = _NC * _NS
_RW = _B // _NW
_GROUPS = _RW // 16
_HROWS = (_B // 128) // _NS
_ZCHUNK = _CPAD // _NS
_NPAIR = 53248


def _vsqrt16(x):
    i = plsc.bitcast(x, jnp.int32)
    y = plsc.bitcast(jnp.int32(0x5F3759DF) - (i >> 1), jnp.float32)
    h = 0.5 * x
    for _ in range(4):
        y = y * (1.5 - (h * y) * y)
    return x * y


def _hist_body(label_hbm, num_hbm, lbl_h, ones_v, zeros_v, numh,
               count_s, hsem):
    cid = lax.axis_index("c")
    sid = lax.axis_index("s")

    def _zbody(i, carry):
        zeros_v[pl.ds(i * 16, 16)] = jnp.zeros((16,), jnp.float32)
        return carry
    lax.fori_loop(0, _ZCHUNK // 16, _zbody, 0)
    for j in range(8):
        ones_v[pl.ds(j * 16, 16)] = jnp.ones((16,), jnp.float32)
    pltpu.sync_copy(zeros_v, count_s.at[pl.ds(sid * _ZCHUNK, _ZCHUNK)])
    plsc.subcore_barrier()

    pltpu.sync_copy(label_hbm.at[pl.ds(sid * _HROWS, _HROWS)], lbl_h)
    adds = [pltpu.async_copy(ones_v, count_s.at[lbl_h.at[j]], hsem, add=True)
            for j in range(_HROWS)]
    for cp in adds:
        cp.wait()
    plsc.subcore_barrier()

    base = cid * (_HROWS // 2)
    gets = [pltpu.async_copy(count_s.at[lbl_h.at[base + c]],
                             numh.at[c], hsem)
            for c in range(_HROWS // 2)]
    for cp in gets:
        cp.wait()
    pltpu.sync_copy(
        numh, num_hbm.at[pl.ds(sid * _HROWS + base, _HROWS // 2)])


def _dist_body(feat_hbm, label_hbm, centers_hbm, num_hbm, out_hbm,
               lbl_d, pidx, cent_v, feat_v, num_v,
               acc_v, sums_v, out_v, partials_s, sem):
    cid = lax.axis_index("c")
    sid = lax.axis_index("s")
    wid = sid * _NC + cid

    pltpu.sync_copy(label_hbm.at[pl.ds(wid * 4, 4)], lbl_d)
    for j in range(4):
        for k in range(8):
            v = lbl_d[j, pl.ds(k * 16, 16)]
            pidx[j, pl.ds(k * 16, 16)] = jnp.where(
                v >= _NPAIR, v - _NPAIR, v)

    copies = []
    for c in range(4):
        copies.append(pltpu.async_copy(
            centers_hbm.at[pidx.at[c]],
            cent_v.at[pl.ds(c * 128, 128)], sem))
    for tc in range(4):
        copies.append(pltpu.async_copy(
            feat_hbm.at[:, pl.ds((wid * 4 + tc) * 128, 128)],
            feat_v.at[tc], sem))
    copies.append(pltpu.async_copy(
        num_hbm.at[pl.ds(wid * 4, 4)], num_v, sem))
    for cp in copies:
        cp.wait()

    iota = lax.iota(jnp.int32, 16)

    def _gbody(g, acc):
        lblv = lbl_d[g // 8, pl.ds((g % 8) * 16, 16)]
        par64 = jnp.where(lblv >= _NPAIR, jnp.int32(64), jnp.int32(0))
        rows16 = g * 16 + iota
        d2 = jnp.zeros((16,), jnp.float32)
        for d in range(_D):
            fv = feat_v[g // 8, d, pl.ds((g % 8) * 16, 16)]
            cv = plsc.load_gather(cent_v, [rows16, par64 + d])
            t = fv - cv
            d2 = d2 + t * t
        num16 = num_v[g // 8, pl.ds((g % 8) * 16, 16)]
        return acc + _vsqrt16(d2) / num16

    acc = lax.fori_loop(0, _GROUPS, _gbody, jnp.zeros((16,), jnp.float32))

    acc_v[pl.ds(0, 16)] = acc
    for j in range(1, 8):
        acc_v[pl.ds(j * 16, 16)] = jnp.zeros((16,), jnp.float32)
    pltpu.sync_copy(acc_v, partials_s.at[sid])
    plsc.subcore_barrier()

    @pl.when(sid == 0)
    def _():
        pltpu.sync_copy(partials_s, sums_v)
        tot = jnp.zeros((16,), jnp.float32)
        for i in range(_NS):
            tot = tot + sums_v[i, pl.ds(0, 16)]
        total = jnp.sum(tot)
        for j in range(8):
            out_v[pl.ds(j * 16, 16)] = jnp.full((16,), total, jnp.float32)
        pltpu.sync_copy(out_v, out_hbm.at[cid])


def _pack_body(lo_ref, hi_ref, o_ref):
    eye = jnp.eye(_D, dtype=jnp.float32)
    dn = (((0,), (0,)), ((), ()))
    lo_t = jax.lax.dot_general(lo_ref[...], eye, dn,
                               preferred_element_type=jnp.float32)
    hi_t = jax.lax.dot_general(hi_ref[...], eye, dn,
                               preferred_element_type=jnp.float32)
    o_ref[...] = jnp.concatenate([lo_t, hi_t], axis=1)


def _pack_centers(centers_t):
    return pl.pallas_call(
        _pack_body,
        out_shape=jax.ShapeDtypeStruct((_NPAIR, 128), jnp.float32),
        grid=(_NPAIR // 4096,),
        in_specs=[
            pl.BlockSpec((_D, 4096), lambda k: (0, k)),
            pl.BlockSpec((_D, 4096),
                         lambda k: (0, jnp.minimum(k + _NPAIR // 4096,
                                                   (_C - 1) // 4096))),
        ],
        out_specs=pl.BlockSpec((4096, 128), lambda k: (k, 0)),
    )(centers_t, centers_t)


def _count_gather(label_r):
    mesh = plsc.VectorSubcoreMesh(core_axis_name="c", subcore_axis_name="s")
    f = pl.kernel(
        _hist_body,
        out_type=jax.ShapeDtypeStruct((_B // 128, 128), jnp.float32),
        mesh=mesh,
        scratch_types=[
            pltpu.VMEM((_HROWS, 128), jnp.int32),
            pltpu.VMEM((128,), jnp.float32),
            pltpu.VMEM((_ZCHUNK,), jnp.float32),
            pltpu.VMEM((_HROWS // 2, 128), jnp.float32),
            pltpu.VMEM_SHARED((_CPAD,), jnp.float32),
            pltpu.SemaphoreType.DMA,
        ],
        compiler_params=pltpu.CompilerParams(
            needs_layout_passes=False, use_tc_tiling_on_sc=True),
    )
    return f(label_r)


def _dist_loss(feat_t, label_r, centers_g, num):
    mesh = plsc.VectorSubcoreMesh(core_axis_name="c", subcore_axis_name="s")
    f = pl.kernel(
        _dist_body,
        out_type=jax.ShapeDtypeStruct((_NC, 128), jnp.float32),
        mesh=mesh,
        scratch_types=[
            pltpu.VMEM((4, 128), jnp.int32),
            pltpu.VMEM((4, 128), jnp.int32),
            pltpu.VMEM((_RW, 128), jnp.float32),
            pltpu.VMEM((4, _D, 128), jnp.float32),
            pltpu.VMEM((4, 128), jnp.float32),
            pltpu.VMEM((128,), jnp.float32),
            pltpu.VMEM((_NS, 128), jnp.float32),
            pltpu.VMEM((128,), jnp.float32),
            pltpu.VMEM_SHARED((_NS, 128), jnp.float32),
            pltpu.SemaphoreType.DMA,
        ],
        compiler_params=pltpu.CompilerParams(
            needs_layout_passes=False, use_tc_tiling_on_sc=True),
    )
    return f(feat_t, label_r, centers_g, num)


@jax.jit
def _center_loss(feat, label, centers):
    feat_t = feat.T
    label_r = label.reshape(_B // 128, 128)
    num = _count_gather(label_r)
    centers_g = _pack_centers(centers.T)
    out = _dist_loss(feat_t, label_r, centers_g, num)
    return (out[0, 0] + out[1, 0]) / jnp.float32(_B)


def kernel(feat, label, centers):
    return _center_loss(feat, label, centers)


# speedup vs baseline: 1.4171x; 1.0061x over previous
import jax
import jax.numpy as jnp
from jax import lax
from jax.experimental import pallas as pl
from jax.experimental.pallas import tpu as pltpu
from jax.experimental.pallas import tpu_sc as plsc

_B = 16384
_D = 64
_C = 100000
_CPAD = 100096
_NC = 2
_NS = 16
_NW = _NC * _NS
_RW = _B // _NW
_GROUPS = _RW // 16
_HROWS = (_B // 128) // _NS
_ZCHUNK = _CPAD // _NS
_NPAIR = 57344


def _vsqrt16(x):
    i = plsc.bitcast(x, jnp.int32)
    y = plsc.bitcast(jnp.int32(0x5F3759DF) - (i >> 1), jnp.float32)
    h = 0.5 * x
    for _ in range(4):
        y = y * (1.5 - (h * y) * y)
    return x * y


def _hist_body(label_hbm, num_hbm, lbl_h, ones_v, zeros_v, numh,
               count_s, hsem):
    cid = lax.axis_index("c")
    sid = lax.axis_index("s")

    def _zbody(i, carry):
        zeros_v[pl.ds(i * 16, 16)] = jnp.zeros((16,), jnp.float32)
        return carry
    lax.fori_loop(0, _ZCHUNK // 16, _zbody, 0)
    for j in range(8):
        ones_v[pl.ds(j * 16, 16)] = jnp.ones((16,), jnp.float32)
    pltpu.sync_copy(zeros_v, count_s.at[pl.ds(sid * _ZCHUNK, _ZCHUNK)])
    plsc.subcore_barrier()

    pltpu.sync_copy(label_hbm.at[pl.ds(sid * _HROWS, _HROWS)], lbl_h)
    adds = [pltpu.async_copy(ones_v, count_s.at[lbl_h.at[j]], hsem, add=True)
            for j in range(_HROWS)]
    for cp in adds:
        cp.wait()
    plsc.subcore_barrier()

    base = cid * (_HROWS // 2)
    gets = [pltpu.async_copy(count_s.at[lbl_h.at[base + c]],
                             numh.at[c], hsem)
            for c in range(_HROWS // 2)]
    for cp in gets:
        cp.wait()
    pltpu.sync_copy(
        numh, num_hbm.at[pl.ds(sid * _HROWS + base, _HROWS // 2)])


def _dist_body(feat_hbm, label_hbm, centers_hbm, num_hbm, out_hbm,
               lbl_d, pidx, cent_v, feat_v, num_v,
               acc_v, sums_v, out_v, partials_s, sem):
    cid = lax.axis_index("c")
    sid = lax.axis_index("s")
    wid = sid * _NC + cid

    pltpu.sync_copy(label_hbm.at[pl.ds(wid * 4, 4)], lbl_d)
    for j in range(4):
        for k in range(8):
            v = lbl_d[j, pl.ds(k * 16, 16)]
            pidx[j, pl.ds(k * 16, 16)] = jnp.where(
                v >= _NPAIR, v - _NPAIR, v)

    copies = []
    for c in range(4):
        copies.append(pltpu.async_copy(
            centers_hbm.at[pidx.at[c]],
            cent_v.at[pl.ds(c * 128, 128)], sem))
    for tc in range(4):
        copies.append(pltpu.async_copy(
            feat_hbm.at[:, pl.ds((wid * 4 + tc) * 128, 128)],
            feat_v.at[tc], sem))
    copies.append(pltpu.async_copy(
        num_hbm.at[pl.ds(wid * 4, 4)], num_v, sem))
    for cp in copies:
        cp.wait()

    iota = lax.iota(jnp.int32, 16)

    def _gbody(g, acc):
        lblv = lbl_d[g // 8, pl.ds((g % 8) * 16, 16)]
        par64 = jnp.where(lblv >= _NPAIR, jnp.int32(64), jnp.int32(0))
        rows16 = g * 16 + iota
        d2 = jnp.zeros((16,), jnp.float32)
        for d in range(_D):
            fv = feat_v[g // 8, d, pl.ds((g % 8) * 16, 16)]
            cv = plsc.load_gather(cent_v, [rows16, par64 + d])
            t = fv - cv
            d2 = d2 + t * t
        num16 = num_v[g // 8, pl.ds((g % 8) * 16, 16)]
        return acc + _vsqrt16(d2) / num16

    acc = lax.fori_loop(0, _GROUPS, _gbody, jnp.zeros((16,), jnp.float32))

    acc_v[pl.ds(0, 16)] = acc
    for j in range(1, 8):
        acc_v[pl.ds(j * 16, 16)] = jnp.zeros((16,), jnp.float32)
    pltpu.sync_copy(acc_v, partials_s.at[sid])
    plsc.subcore_barrier()

    @pl.when(sid == 0)
    def _():
        pltpu.sync_copy(partials_s, sums_v)
        tot = jnp.zeros((16,), jnp.float32)
        for i in range(_NS):
            tot = tot + sums_v[i, pl.ds(0, 16)]
        total = jnp.sum(tot)
        for j in range(8):
            out_v[pl.ds(j * 16, 16)] = jnp.full((16,), total, jnp.float32)
        pltpu.sync_copy(out_v, out_hbm.at[cid])


def _pack_body(lo_ref, hi_ref, o_ref):
    eye = jnp.eye(_D, dtype=jnp.float32)
    dn = (((0,), (0,)), ((), ()))
    lo_t = jax.lax.dot_general(lo_ref[...], eye, dn,
                               preferred_element_type=jnp.float32)
    hi_t = jax.lax.dot_general(hi_ref[...], eye, dn,
                               preferred_element_type=jnp.float32)
    o_ref[...] = jnp.concatenate([lo_t, hi_t], axis=1)


def _pack_centers(centers_t):
    return pl.pallas_call(
        _pack_body,
        out_shape=jax.ShapeDtypeStruct((_NPAIR, 128), jnp.float32),
        grid=(_NPAIR // 8192,),
        in_specs=[
            pl.BlockSpec((_D, 8192), lambda k: (0, k)),
            pl.BlockSpec((_D, 8192),
                         lambda k: (0, jnp.minimum(k + _NPAIR // 8192,
                                                   (_C - 1) // 8192))),
        ],
        out_specs=pl.BlockSpec((8192, 128), lambda k: (k, 0)),
    )(centers_t, centers_t)


def _count_gather(label_r):
    mesh = plsc.VectorSubcoreMesh(core_axis_name="c", subcore_axis_name="s")
    f = pl.kernel(
        _hist_body,
        out_type=jax.ShapeDtypeStruct((_B // 128, 128), jnp.float32),
        mesh=mesh,
        scratch_types=[
            pltpu.VMEM((_HROWS, 128), jnp.int32),
            pltpu.VMEM((128,), jnp.float32),
            pltpu.VMEM((_ZCHUNK,), jnp.float32),
            pltpu.VMEM((_HROWS // 2, 128), jnp.float32),
            pltpu.VMEM_SHARED((_CPAD,), jnp.float32),
            pltpu.SemaphoreType.DMA,
        ],
        compiler_params=pltpu.CompilerParams(
            needs_layout_passes=False, use_tc_tiling_on_sc=True),
    )
    return f(label_r)


def _dist_loss(feat_t, label_r, centers_g, num):
    mesh = plsc.VectorSubcoreMesh(core_axis_name="c", subcore_axis_name="s")
    f = pl.kernel(
        _dist_body,
        out_type=jax.ShapeDtypeStruct((_NC, 128), jnp.float32),
        mesh=mesh,
        scratch_types=[
            pltpu.VMEM((4, 128), jnp.int32),
            pltpu.VMEM((4, 128), jnp.int32),
            pltpu.VMEM((_RW, 128), jnp.float32),
            pltpu.VMEM((4, _D, 128), jnp.float32),
            pltpu.VMEM((4, 128), jnp.float32),
            pltpu.VMEM((128,), jnp.float32),
            pltpu.VMEM((_NS, 128), jnp.float32),
            pltpu.VMEM((128,), jnp.float32),
            pltpu.VMEM_SHARED((_NS, 128), jnp.float32),
            pltpu.SemaphoreType.DMA,
        ],
        compiler_params=pltpu.CompilerParams(
            needs_layout_passes=False, use_tc_tiling_on_sc=True),
    )
    return f(feat_t, label_r, centers_g, num)


@jax.jit
def _center_loss(feat, label, centers):
    feat_t = feat.T
    label_r = label.reshape(_B // 128, 128)
    num = _count_gather(label_r)
    centers_g = _pack_centers(centers.T)
    out = _dist_loss(feat_t, label_r, centers_g, num)
    return (out[0, 0] + out[1, 0]) / jnp.float32(_B)


def kernel(feat, label, centers):
    return _center_loss(feat, label, centers)
